# baseline (speedup 1.0000x reference)
import functools

import jax
import jax.numpy as jnp
from jax import lax
from jax.experimental import pallas as pl
from jax.experimental.pallas import tpu as pltpu
from jax.experimental.pallas import tpu_sc as plsc

N = 10000
E = 320000
D = 128
HD = 64

NC = 2
NS = 16
NW = NC * NS

CH = 128
NB = 4
NCH = 80
NPASS = NCH // NB
E_PAD = NW * NCH * CH

N_ACC = 10240
RPT = N_ACC // NS
RCH = 128
NRC = RPT // RCH

_mesh = plsc.VectorSubcoreMesh(core_axis_name="c", subcore_axis_name="s",
                               num_cores=NC, num_subcores=NS)
_sc_params = pltpu.CompilerParams(use_tc_tiling_on_sc=False)


@functools.partial(
    pl.kernel,
    out_type=jax.ShapeDtypeStruct((NC, N_ACC, 16), jnp.float32),
    mesh=_mesh,
    scratch_types=[
        pltpu.VMEM((NCH, CH), jnp.int32),
        pltpu.VMEM((CH, 16), jnp.float32),
        pltpu.VMEM((RCH, 16), jnp.float32),
        pltpu.VMEM_SHARED((N_ACC, 16), jnp.float32),
        pltpu.SemaphoreType.DMA,
    ],
    compiler_params=_sc_params,
)
def _sc_count(zeros_hbm, ones_hbm, dst_hbm, out_hbm,
              dst_v, ones_v, buf_v, acc_sh, sem):
    cid = lax.axis_index("c")
    sid = lax.axis_index("s")
    wid = cid * NS + sid
    pltpu.sync_copy(zeros_hbm, buf_v)
    for k in range(NRC):
        r0 = sid * RPT + k * RCH
        pltpu.sync_copy(buf_v, acc_sh.at[pl.ds(r0, RCH)])
    pltpu.sync_copy(ones_hbm, ones_v)
    pltpu.sync_copy(dst_hbm.at[wid], dst_v)
    plsc.subcore_barrier()
    def body(j, carry):
        pltpu.sync_copy(ones_v, acc_sh.at[dst_v.at[j]], add=True)
        return carry
    lax.fori_loop(0, NCH, body, 0)
    plsc.subcore_barrier()
    for k in range(NRC):
        r0 = sid * RPT + k * RCH
        pltpu.sync_copy(acc_sh.at[pl.ds(r0, RCH)], buf_v)
        pltpu.sync_copy(buf_v, out_hbm.at[cid, pl.ds(r0, RCH)])


@functools.partial(
    pl.kernel,
    out_type=[jax.ShapeDtypeStruct((NC, N_ACC, HD), jnp.float32),
              jax.ShapeDtypeStruct((NC, N_ACC, HD), jnp.float32)],
    mesh=_mesh,
    scratch_types=[
        pltpu.VMEM((NCH, CH), jnp.int32),
        pltpu.VMEM((NCH, CH), jnp.int32),
        pltpu.VMEM((NB * CH, HD), jnp.float32),
        pltpu.VMEM_SHARED((N_ACC, HD), jnp.float32),
        pltpu.SemaphoreType.DMA((NB,)),
    ],
    compiler_params=_sc_params,
)
def _sc_segsum(ga_hbm, gb_hbm, src_hbm, dst_hbm, outa_hbm, outb_hbm,
               src_v, dst_v, ring, acc_sh, sem_g):
    cid = lax.axis_index("c")
    sid = lax.axis_index("s")
    wid = cid * NS + sid
    pltpu.sync_copy(src_hbm.at[wid], src_v)
    pltpu.sync_copy(dst_hbm.at[wid], dst_v)
    for g_hbm, out_hbm in ((ga_hbm, outa_hbm), (gb_hbm, outb_hbm)):
        for k in range(NRC):
            r0 = sid * RPT + k * RCH
            pltpu.sync_copy(g_hbm.at[pl.ds(r0, RCH)], ring.at[pl.ds(0, RCH)])
            pltpu.sync_copy(ring.at[pl.ds(0, RCH)], acc_sh.at[pl.ds(r0, RCH)])
        plsc.subcore_barrier()
        for b in range(NB):
            pltpu.async_copy(g_hbm.at[src_v.at[b]],
                             ring.at[pl.ds(b * CH, CH)], sem_g.at[b])

        def body(jj, carry):
            j0 = jj * NB
            for b in range(NB):
                pltpu.make_async_copy(g_hbm.at[src_v.at[0]],
                                      ring.at[pl.ds(b * CH, CH)],
                                      sem_g.at[b]).wait()
                pltpu.sync_copy(ring.at[pl.ds(b * CH, CH)],
                                acc_sh.at[dst_v.at[j0 + b]], add=True)
                jn = jnp.minimum(j0 + NB + b, NCH - 1)
                pltpu.async_copy(g_hbm.at[src_v.at[jn]],
                                 ring.at[pl.ds(b * CH, CH)], sem_g.at[b])
            return carry

        lax.fori_loop(0, NPASS, body, 0)
        for b in range(NB):
            pltpu.make_async_copy(g_hbm.at[src_v.at[0]],
                                  ring.at[pl.ds(b * CH, CH)],
                                  sem_g.at[b]).wait()
        plsc.subcore_barrier()
        for k in range(NRC):
            r0 = sid * RPT + k * RCH
            pltpu.sync_copy(acc_sh.at[pl.ds(r0, RCH)], ring.at[pl.ds(0, RCH)])
            pltpu.sync_copy(ring.at[pl.ds(0, RCH)],
                            out_hbm.at[cid, pl.ds(r0, RCH)])


RB = 1280
_GRID = N_ACC // RB


def _tc_prep_body(x_ref, cnt_ref, w0_ref, wr_ref, br_ref,
                  ga_ref, gb_ref, res_ref, dinv_ref):
    deg = cnt_ref[0, :, 0:1] + cnt_ref[1, :, 0:1] + 1.0
    dinv = lax.rsqrt(deg)
    x = x_ref[...]
    hw = jnp.dot(x, w0_ref[...], preferred_element_type=jnp.float32,
                 precision=lax.Precision.HIGHEST)
    g = hw * dinv
    ga_ref[...] = g[:, :HD]
    gb_ref[...] = g[:, HD:]
    res_ref[...] = jnp.dot(x, wr_ref[...], preferred_element_type=jnp.float32,
                           precision=lax.Precision.HIGHEST) + br_ref[...]
    dinv_ref[...] = jnp.broadcast_to(dinv, (RB, 16))


def _tc_prep(x_pad, cntp, W0, Wr, br2):
    return pl.pallas_call(
        _tc_prep_body,
        grid=(_GRID,),
        in_specs=[
            pl.BlockSpec((RB, D), lambda i: (i, 0)),
            pl.BlockSpec((NC, RB, 16), lambda i: (0, i, 0)),
            pl.BlockSpec((D, D), lambda i: (0, 0)),
            pl.BlockSpec((D, D), lambda i: (0, 0)),
            pl.BlockSpec((1, D), lambda i: (0, 0)),
        ],
        out_specs=[
            pl.BlockSpec((RB, HD), lambda i: (i, 0)),
            pl.BlockSpec((RB, HD), lambda i: (i, 0)),
            pl.BlockSpec((RB, D), lambda i: (i, 0)),
            pl.BlockSpec((RB, 16), lambda i: (i, 0)),
        ],
        out_shape=[
            jax.ShapeDtypeStruct((N_ACC, HD), jnp.float32),
            jax.ShapeDtypeStruct((N_ACC, HD), jnp.float32),
            jax.ShapeDtypeStruct((N_ACC, D), jnp.float32),
            jax.ShapeDtypeStruct((N_ACC, 16), jnp.float32),
        ],
    )(x_pad, cntp, W0, Wr, br2)


def _tc_mid_body(pa_ref, pb_ref, ga_ref, gb_ref, dinv_ref, res_ref, b_ref,
                 w_ref, gna_ref, gnb_ref, *, use_res):
    dinv = dinv_ref[:, 0:1]
    sa = pa_ref[0] + pa_ref[1] - ga_ref[...]
    sb = pb_ref[0] + pb_ref[1] - gb_ref[...]
    s = jnp.concatenate([sa, sb], axis=1)
    h = jnp.maximum(dinv * s + b_ref[...], 0.0)
    if use_res:
        h = h + res_ref[...]
    hw = jnp.dot(h, w_ref[...], preferred_element_type=jnp.float32,
                 precision=lax.Precision.HIGHEST)
    g = hw * dinv
    gna_ref[...] = g[:, :HD]
    gnb_ref[...] = g[:, HD:]


def _tc_mid(pa, pb, ga, gb, dinv16, res, b2, Wn, use_res):
    return pl.pallas_call(
        functools.partial(_tc_mid_body, use_res=use_res),
        grid=(_GRID,),
        in_specs=[
            pl.BlockSpec((NC, RB, HD), lambda i: (0, i, 0)),
            pl.BlockSpec((NC, RB, HD), lambda i: (0, i, 0)),
            pl.BlockSpec((RB, HD), lambda i: (i, 0)),
            pl.BlockSpec((RB, HD), lambda i: (i, 0)),
            pl.BlockSpec((RB, 16), lambda i: (i, 0)),
            pl.BlockSpec((RB, D), lambda i: (i, 0)),
            pl.BlockSpec((1, D), lambda i: (0, 0)),
            pl.BlockSpec((D, D), lambda i: (0, 0)),
        ],
        out_specs=[
            pl.BlockSpec((RB, HD), lambda i: (i, 0)),
            pl.BlockSpec((RB, HD), lambda i: (i, 0)),
        ],
        out_shape=[
            jax.ShapeDtypeStruct((N_ACC, HD), jnp.float32),
            jax.ShapeDtypeStruct((N_ACC, HD), jnp.float32),
        ],
    )(pa, pb, ga, gb, dinv16, res, b2, Wn)


def _tc_final_body(pa_ref, pb_ref, ga_ref, gb_ref, dinv_ref, b_ref, h_ref):
    dinv = dinv_ref[:, 0:1]
    sa = pa_ref[0] + pa_ref[1] - ga_ref[...]
    sb = pb_ref[0] + pb_ref[1] - gb_ref[...]
    s = jnp.concatenate([sa, sb], axis=1)
    h_ref[...] = jnp.maximum(dinv * s + b_ref[...], 0.0)


def _tc_final(pa, pb, ga, gb, dinv16, b2):
    return pl.pallas_call(
        _tc_final_body,
        grid=(_GRID,),
        in_specs=[
            pl.BlockSpec((NC, RB, HD), lambda i: (0, i, 0)),
            pl.BlockSpec((NC, RB, HD), lambda i: (0, i, 0)),
            pl.BlockSpec((RB, HD), lambda i: (i, 0)),
            pl.BlockSpec((RB, HD), lambda i: (i, 0)),
            pl.BlockSpec((RB, 16), lambda i: (i, 0)),
            pl.BlockSpec((1, D), lambda i: (0, 0)),
        ],
        out_specs=pl.BlockSpec((RB, D), lambda i: (i, 0)),
        out_shape=jax.ShapeDtypeStruct((N_ACC, D), jnp.float32),
    )(pa, pb, ga, gb, dinv16, b2)


def kernel(x, edge_index, W0, b0, W1, b1, W2, b2, Wr, br):
    src = edge_index[0].astype(jnp.int32)
    dst = edge_index[1].astype(jnp.int32)
    npad = E_PAD - E
    pad_i = jnp.arange(npad, dtype=jnp.int32)
    src_p = jnp.concatenate([src, pad_i % N])
    dst_p = jnp.concatenate([dst, N + pad_i % (N_ACC - N)])
    src_r = src_p.reshape(NW, NCH, CH)
    dst_r = dst_p.reshape(NW, NCH, CH)

    x_pad = jnp.pad(x, ((0, N_ACC - N), (0, 0)))
    zeros16 = jnp.zeros((RCH, 16), jnp.float32)
    ones16 = jnp.ones((CH, 16), jnp.float32)
    b0_2 = b0.reshape(1, D)
    b1_2 = b1.reshape(1, D)
    b2_2 = b2.reshape(1, D)
    br_2 = br.reshape(1, D)

    cntp = _sc_count(zeros16, ones16, dst_r)
    ga, gb, res, dinv16 = _tc_prep(x_pad, cntp, W0, Wr, br_2)
    pa, pb = _sc_segsum(ga, gb, src_r, dst_r)
    ga, gb = _tc_mid(pa, pb, ga, gb, dinv16, res, b0_2, W1, use_res=True)
    pa, pb = _sc_segsum(ga, gb, src_r, dst_r)
    ga, gb = _tc_mid(pa, pb, ga, gb, dinv16, res, b1_2, W2, use_res=False)
    pa, pb = _sc_segsum(ga, gb, src_r, dst_r)
    h = _tc_final(pa, pb, ga, gb, dinv16, b2_2)
    return h[:N]


# speedup vs baseline: 1.0509x; 1.0509x over previous
import functools

import jax
import jax.numpy as jnp
from jax import lax
from jax.experimental import pallas as pl
from jax.experimental.pallas import tpu as pltpu
from jax.experimental.pallas import tpu_sc as plsc

N = 10000
E = 320000
D = 128
HD = 64

NC = 2
NS = 16
NW = NC * NS

CH = 128
NB = 4
NCH = 80
NPASS = NCH // NB
E_PAD = NW * NCH * CH

N_ACC = 10240
RPT = N_ACC // NS
RCH = 128
NRC = RPT // RCH

_mesh = plsc.VectorSubcoreMesh(core_axis_name="c", subcore_axis_name="s",
                               num_cores=NC, num_subcores=NS)
_sc_params = pltpu.CompilerParams(use_tc_tiling_on_sc=False)


@functools.partial(
    pl.kernel,
    out_type=jax.ShapeDtypeStruct((NC, N_ACC, 16), jnp.float32),
    mesh=_mesh,
    scratch_types=[
        pltpu.VMEM((NCH, CH), jnp.int32),
        pltpu.VMEM((CH, 16), jnp.float32),
        pltpu.VMEM((RCH, 16), jnp.float32),
        pltpu.VMEM_SHARED((N_ACC, 16), jnp.float32),
        pltpu.SemaphoreType.DMA,
    ],
    compiler_params=_sc_params,
)
def _sc_count(zeros_hbm, ones_hbm, dst_hbm, out_hbm,
              dst_v, ones_v, buf_v, acc_sh, sem):
    cid = lax.axis_index("c")
    sid = lax.axis_index("s")
    wid = cid * NS + sid
    pltpu.sync_copy(zeros_hbm, buf_v)
    for k in range(NRC):
        r0 = sid * RPT + k * RCH
        pltpu.sync_copy(buf_v, acc_sh.at[pl.ds(r0, RCH)])
    pltpu.sync_copy(ones_hbm, ones_v)
    pltpu.sync_copy(dst_hbm.at[wid], dst_v)
    plsc.subcore_barrier()
    def body(j, carry):
        pltpu.sync_copy(ones_v, acc_sh.at[dst_v.at[j]], add=True)
        return carry
    lax.fori_loop(0, NCH, body, 0)
    plsc.subcore_barrier()
    for k in range(NRC):
        r0 = sid * RPT + k * RCH
        pltpu.sync_copy(acc_sh.at[pl.ds(r0, RCH)], buf_v)
        pltpu.sync_copy(buf_v, out_hbm.at[cid, pl.ds(r0, RCH)])


@functools.partial(
    pl.kernel,
    out_type=[jax.ShapeDtypeStruct((NC, N_ACC, HD), jnp.float32),
              jax.ShapeDtypeStruct((NC, N_ACC, HD), jnp.float32)],
    mesh=_mesh,
    scratch_types=[
        pltpu.VMEM((NCH, CH), jnp.int32),
        pltpu.VMEM((NCH, CH), jnp.int32),
        pltpu.VMEM((NB * CH, HD), jnp.float32),
        pltpu.VMEM_SHARED((N_ACC, HD), jnp.float32),
        pltpu.SemaphoreType.DMA((NB,)),
    ],
    compiler_params=_sc_params,
)
def _sc_segsum(ga_hbm, gb_hbm, src_hbm, dst_hbm, outa_hbm, outb_hbm,
               src_v, dst_v, ring, acc_sh, sem_g):
    cid = lax.axis_index("c")
    sid = lax.axis_index("s")
    wid = cid * NS + sid
    pltpu.sync_copy(src_hbm.at[wid], src_v)
    pltpu.sync_copy(dst_hbm.at[wid], dst_v)
    for g_hbm, out_hbm in ((ga_hbm, outa_hbm), (gb_hbm, outb_hbm)):
        r0 = sid * RPT
        pltpu.sync_copy(g_hbm.at[pl.ds(r0, RPT)], acc_sh.at[pl.ds(r0, RPT)])
        plsc.subcore_barrier()
        for b in range(NB):
            pltpu.async_copy(g_hbm.at[src_v.at[b]],
                             ring.at[pl.ds(b * CH, CH)], sem_g.at[b])

        def body(jj, carry):
            j0 = jj * NB
            for b in range(NB):
                pltpu.make_async_copy(g_hbm.at[src_v.at[0]],
                                      ring.at[pl.ds(b * CH, CH)],
                                      sem_g.at[b]).wait()
                pltpu.sync_copy(ring.at[pl.ds(b * CH, CH)],
                                acc_sh.at[dst_v.at[j0 + b]], add=True)
                jn = jnp.minimum(j0 + NB + b, NCH - 1)
                pltpu.async_copy(g_hbm.at[src_v.at[jn]],
                                 ring.at[pl.ds(b * CH, CH)], sem_g.at[b])
            return carry

        lax.fori_loop(0, NPASS, body, 0)
        for b in range(NB):
            pltpu.make_async_copy(g_hbm.at[src_v.at[0]],
                                  ring.at[pl.ds(b * CH, CH)],
                                  sem_g.at[b]).wait()
        plsc.subcore_barrier()
        r0 = sid * RPT
        pltpu.sync_copy(acc_sh.at[pl.ds(r0, RPT)],
                        out_hbm.at[cid, pl.ds(r0, RPT)])


RB = 1280
_GRID = N_ACC // RB


def _tc_prep_body(x_ref, cnt_ref, w0_ref, wr_ref, br_ref,
                  ga_ref, gb_ref, res_ref, dinv_ref):
    deg = cnt_ref[0, :, 0:1] + cnt_ref[1, :, 0:1] + 1.0
    dinv = lax.rsqrt(deg)
    x = x_ref[...]
    hw = jnp.dot(x, w0_ref[...], preferred_element_type=jnp.float32,
                 precision=lax.Precision.HIGHEST)
    g = hw * dinv
    ga_ref[...] = g[:, :HD]
    gb_ref[...] = g[:, HD:]
    res_ref[...] = jnp.dot(x, wr_ref[...], preferred_element_type=jnp.float32,
                           precision=lax.Precision.HIGHEST) + br_ref[...]
    dinv_ref[...] = jnp.broadcast_to(dinv, (RB, 16))


def _tc_prep(x_pad, cntp, W0, Wr, br2):
    return pl.pallas_call(
        _tc_prep_body,
        grid=(_GRID,),
        in_specs=[
            pl.BlockSpec((RB, D), lambda i: (i, 0)),
            pl.BlockSpec((NC, RB, 16), lambda i: (0, i, 0)),
            pl.BlockSpec((D, D), lambda i: (0, 0)),
            pl.BlockSpec((D, D), lambda i: (0, 0)),
            pl.BlockSpec((1, D), lambda i: (0, 0)),
        ],
        out_specs=[
            pl.BlockSpec((RB, HD), lambda i: (i, 0)),
            pl.BlockSpec((RB, HD), lambda i: (i, 0)),
            pl.BlockSpec((RB, D), lambda i: (i, 0)),
            pl.BlockSpec((RB, 16), lambda i: (i, 0)),
        ],
        out_shape=[
            jax.ShapeDtypeStruct((N_ACC, HD), jnp.float32),
            jax.ShapeDtypeStruct((N_ACC, HD), jnp.float32),
            jax.ShapeDtypeStruct((N_ACC, D), jnp.float32),
            jax.ShapeDtypeStruct((N_ACC, 16), jnp.float32),
        ],
    )(x_pad, cntp, W0, Wr, br2)


def _tc_mid_body(pa_ref, pb_ref, ga_ref, gb_ref, dinv_ref, res_ref, b_ref,
                 w_ref, gna_ref, gnb_ref, *, use_res):
    dinv = dinv_ref[:, 0:1]
    sa = pa_ref[0] + pa_ref[1] - ga_ref[...]
    sb = pb_ref[0] + pb_ref[1] - gb_ref[...]
    s = jnp.concatenate([sa, sb], axis=1)
    h = jnp.maximum(dinv * s + b_ref[...], 0.0)
    if use_res:
        h = h + res_ref[...]
    hw = jnp.dot(h, w_ref[...], preferred_element_type=jnp.float32,
                 precision=lax.Precision.HIGHEST)
    g = hw * dinv
    gna_ref[...] = g[:, :HD]
    gnb_ref[...] = g[:, HD:]


def _tc_mid(pa, pb, ga, gb, dinv16, res, b2, Wn, use_res):
    return pl.pallas_call(
        functools.partial(_tc_mid_body, use_res=use_res),
        grid=(_GRID,),
        in_specs=[
            pl.BlockSpec((NC, RB, HD), lambda i: (0, i, 0)),
            pl.BlockSpec((NC, RB, HD), lambda i: (0, i, 0)),
            pl.BlockSpec((RB, HD), lambda i: (i, 0)),
            pl.BlockSpec((RB, HD), lambda i: (i, 0)),
            pl.BlockSpec((RB, 16), lambda i: (i, 0)),
            pl.BlockSpec((RB, D), lambda i: (i, 0)),
            pl.BlockSpec((1, D), lambda i: (0, 0)),
            pl.BlockSpec((D, D), lambda i: (0, 0)),
        ],
        out_specs=[
            pl.BlockSpec((RB, HD), lambda i: (i, 0)),
            pl.BlockSpec((RB, HD), lambda i: (i, 0)),
        ],
        out_shape=[
            jax.ShapeDtypeStruct((N_ACC, HD), jnp.float32),
            jax.ShapeDtypeStruct((N_ACC, HD), jnp.float32),
        ],
    )(pa, pb, ga, gb, dinv16, res, b2, Wn)


def _tc_final_body(pa_ref, pb_ref, ga_ref, gb_ref, dinv_ref, b_ref, h_ref):
    dinv = dinv_ref[:, 0:1]
    sa = pa_ref[0] + pa_ref[1] - ga_ref[...]
    sb = pb_ref[0] + pb_ref[1] - gb_ref[...]
    s = jnp.concatenate([sa, sb], axis=1)
    h_ref[...] = jnp.maximum(dinv * s + b_ref[...], 0.0)


def _tc_final(pa, pb, ga, gb, dinv16, b2):
    return pl.pallas_call(
        _tc_final_body,
        grid=(_GRID,),
        in_specs=[
            pl.BlockSpec((NC, RB, HD), lambda i: (0, i, 0)),
            pl.BlockSpec((NC, RB, HD), lambda i: (0, i, 0)),
            pl.BlockSpec((RB, HD), lambda i: (i, 0)),
            pl.BlockSpec((RB, HD), lambda i: (i, 0)),
            pl.BlockSpec((RB, 16), lambda i: (i, 0)),
            pl.BlockSpec((1, D), lambda i: (0, 0)),
        ],
        out_specs=pl.BlockSpec((RB, D), lambda i: (i, 0)),
        out_shape=jax.ShapeDtypeStruct((N_ACC, D), jnp.float32),
    )(pa, pb, ga, gb, dinv16, b2)


def kernel(x, edge_index, W0, b0, W1, b1, W2, b2, Wr, br):
    src = edge_index[0].astype(jnp.int32)
    dst = edge_index[1].astype(jnp.int32)
    npad = E_PAD - E
    pad_i = jnp.arange(npad, dtype=jnp.int32)
    src_p = jnp.concatenate([src, pad_i % N])
    dst_p = jnp.concatenate([dst, N + pad_i % (N_ACC - N)])
    src_r = src_p.reshape(NW, NCH, CH)
    dst_r = dst_p.reshape(NW, NCH, CH)

    x_pad = jnp.pad(x, ((0, N_ACC - N), (0, 0)))
    zeros16 = jnp.zeros((RCH, 16), jnp.float32)
    ones16 = jnp.ones((CH, 16), jnp.float32)
    b0_2 = b0.reshape(1, D)
    b1_2 = b1.reshape(1, D)
    b2_2 = b2.reshape(1, D)
    br_2 = br.reshape(1, D)

    cntp = _sc_count(zeros16, ones16, dst_r)
    ga, gb, res, dinv16 = _tc_prep(x_pad, cntp, W0, Wr, br_2)
    pa, pb = _sc_segsum(ga, gb, src_r, dst_r)
    ga, gb = _tc_mid(pa, pb, ga, gb, dinv16, res, b0_2, W1, use_res=True)
    pa, pb = _sc_segsum(ga, gb, src_r, dst_r)
    ga, gb = _tc_mid(pa, pb, ga, gb, dinv16, res, b1_2, W2, use_res=False)
    pa, pb = _sc_segsum(ga, gb, src_r, dst_r)
    h = _tc_final(pa, pb, ga, gb, dinv16, b2_2)
    return h[:N]
